# Initial kernel scaffold; baseline (speedup 1.0000x reference)
import jax
import jax.numpy as jnp
from jax.experimental import pallas as pl


def kernel(x, edge_index, W1, att_src1, att_dst1, b1, W2, att_src2, att_dst2, b2):
    raise NotImplementedError("write your pallas kernel here")


# speedup vs baseline: 1.0952x; 1.0952x over previous
import functools

import jax
import jax.numpy as jnp
from jax.experimental import pallas as pl


def _mm_body(x_ref, w_ref, o_ref):
    o_ref[...] = jnp.dot(x_ref[...], w_ref[...],
                         preferred_element_type=jnp.float32)


def _matmul(x, w, tn=400):
    n, k = x.shape
    k2, m = w.shape
    grid = n // tn
    return pl.pallas_call(
        _mm_body,
        grid=(grid,),
        in_specs=[
            pl.BlockSpec((tn, k), lambda i: (i, 0)),
            pl.BlockSpec((k, m), lambda i: (0, 0)),
        ],
        out_specs=pl.BlockSpec((tn, m), lambda i: (i, 0)),
        out_shape=jax.ShapeDtypeStruct((n, m), jnp.float32),
    )(x, w)


def _gat_layer(x, W, att_src, att_dst, bias, src, dst, heads, out_ch):
    n = x.shape[0]
    h = _matmul(x, W).reshape(n, heads, out_ch)
    a_src = (h * att_src[None]).sum(-1)
    a_dst = (h * att_dst[None]).sum(-1)
    alpha = a_src[src] + a_dst[dst]
    alpha = jnp.where(alpha >= 0, alpha, 0.2 * alpha)
    w = jnp.exp(alpha)
    denom = jax.ops.segment_sum(w, dst, num_segments=n)
    coef = w / (denom[dst] + 1e-16)
    msg = h[src] * coef[:, :, None]
    out = jax.ops.segment_sum(msg, dst, num_segments=n)
    return out.reshape(n, heads * out_ch) + bias


def kernel(x, edge_index, W1, att_src1, att_dst1, b1,
           W2, att_src2, att_dst2, b2):
    src = edge_index[:, 0].astype(jnp.int32)
    dst = edge_index[:, 1].astype(jnp.int32)
    x1 = jax.nn.elu(_gat_layer(x, W1, att_src1, att_dst1, b1,
                               src, dst, 8, 128))
    out = _gat_layer(x1, W2, att_src2, att_dst2, b2, src, dst, 1, 256)
    return out


# speedup vs baseline: 15.8353x; 14.4589x over previous
import functools

import jax
import jax.numpy as jnp
from jax import lax
from jax.experimental import pallas as pl
from jax.experimental.pallas import tpu as pltpu
from jax.experimental.pallas import tpu_sc as plsc

_N = 10000
_E = 320000
_NP = 10240
_RPT = _NP // 16
_EPB = _E // 16
_EB = 800
_NB = _EPB // _EB
_CH = 80
_NCH = _EB // _CH


def _mm_body(x_ref, w_ref, a_ref, h_ref, o_ref):
    h = jnp.dot(x_ref[...], w_ref[...], preferred_element_type=jnp.float32)
    h_ref[...] = h
    o_ref[...] = jnp.dot(h, a_ref[...], preferred_element_type=jnp.float32)


def _proj1(x, W1, A1):
    tn = 400
    return pl.pallas_call(
        _mm_body,
        grid=(_N // tn,),
        in_specs=[
            pl.BlockSpec((tn, 128), lambda i: (i, 0)),
            pl.BlockSpec((128, 1024), lambda i: (0, 0)),
            pl.BlockSpec((1024, 16), lambda i: (0, 0)),
        ],
        out_specs=[
            pl.BlockSpec((tn, 1024), lambda i: (i, 0)),
            pl.BlockSpec((tn, 16), lambda i: (i, 0)),
        ],
        out_shape=[
            jax.ShapeDtypeStruct((_N, 1024), jnp.float32),
            jax.ShapeDtypeStruct((_N, 16), jnp.float32),
        ],
    )(x, W1, A1)


def _mm2_body(o1_ref, b1_ref, w2_ref, a2_ref, h2_ref, ao_ref):
    acc = jnp.zeros((o1_ref.shape[1], 256), jnp.float32)
    for hh in range(8):
        v = o1_ref[hh] + b1_ref[hh]
        xh = jnp.where(v > 0, v, jnp.exp(v) - 1.0)
        acc += jnp.dot(xh, w2_ref[hh], preferred_element_type=jnp.float32)
    h2_ref[...] = acc
    ao_ref[...] = jnp.dot(acc, a2_ref[...], preferred_element_type=jnp.float32)


def _proj2(out1h, b1r, W2r, A2):
    tn = 400
    return pl.pallas_call(
        _mm2_body,
        grid=(_N // tn,),
        in_specs=[
            pl.BlockSpec((8, tn, 128), lambda i: (0, i, 0)),
            pl.BlockSpec((8, 128), lambda i: (0, 0)),
            pl.BlockSpec((8, 128, 256), lambda i: (0, 0, 0)),
            pl.BlockSpec((256, 8), lambda i: (0, 0)),
        ],
        out_specs=[
            pl.BlockSpec((tn, 256), lambda i: (i, 0)),
            pl.BlockSpec((tn, 8), lambda i: (i, 0)),
        ],
        out_shape=[
            jax.ShapeDtypeStruct((_N, 256), jnp.float32),
            jax.ShapeDtypeStruct((_N, 8), jnp.float32),
        ],
    )(out1h, b1r, W2r, A2)


_GDN = lax.GatherDimensionNumbers(
    offset_dims=(), collapsed_slice_dims=(0,), start_index_map=(0,))


def _lane(cv, l):
    idx = jnp.broadcast_to(l, (16, 1)).astype(jnp.int32)
    return lax.gather(cv, idx, _GDN, (1,),
                      mode=lax.GatherScatterMode.PROMISE_IN_BOUNDS)


def _edge_w(asrc_t, adst_t, sv, dv):
    av = plsc.load_gather(asrc_t, [sv]) + plsc.load_gather(adst_t, [dv])
    av = jnp.where(av >= 0, av, av * 0.2)
    return jnp.exp(av)


def _stats_pass(eb, src_h, dst_h, src_c, dst_c, asrc_t, adst_t, den_t):
    def blk(b, c):
        o = eb + b * _EB
        pltpu.sync_copy(src_h.at[pl.ds(o, _EB)], src_c)
        pltpu.sync_copy(dst_h.at[pl.ds(o, _EB)], dst_c)

        def grp(i, c2):
            sl = pl.ds(i * 16, 16)
            dv = dst_c[sl]
            w = _edge_w(asrc_t, adst_t, src_c[sl], dv)
            plsc.addupdate_scatter(den_t, [dv], w)
            return c2

        lax.fori_loop(0, _EB // 16, grp, 0)
        return c

    lax.fori_loop(0, _NB, blk, 0)


def _reduce_denom(cid, sid, den_t, red_b, redo_c, dscr_h, sh_den):
    pltpu.sync_copy(den_t, dscr_h.at[cid, sid])
    plsc.subcore_barrier()
    nb = sid * _RPT
    for j in range(5):
        pltpu.sync_copy(dscr_h.at[cid, :, pl.ds(nb + j * 128, 128)], red_b)

        def red(i, c2):
            sl = pl.ds(i * 16, 16)
            acc = red_b[0, sl]
            for r in range(1, 16):
                acc = acc + red_b[r, sl]
            redo_c[sl] = acc
            return c2

        lax.fori_loop(0, 8, red, 0)
        pltpu.sync_copy(redo_c, sh_den.at[pl.ds(nb + j * 128, 128)])
    plsc.subcore_barrier()
    pltpu.sync_copy(sh_den, den_t)


def _spmm_pass(eb, head_idx, mult, src_h, dst_h, src_c, dst_c, asrc_t,
               adst_t, den_t, idx_b, dsti_b, rows_b, tab_h, acc_sh, sem):
    def blk(b, c):
        o = eb + b * _EB
        pltpu.sync_copy(src_h.at[pl.ds(o, _EB)], src_c)
        pltpu.sync_copy(dst_h.at[pl.ds(o, _EB)], dst_c)

        def chunk(i, c2):
            cb = i * _CH

            def prep(g, c3):
                sl = pl.ds(g * 16, 16)
                bsl = pl.ds(cb + g * 16, 16)
                idx_b[sl] = src_c[bsl] * mult + head_idx
                dsti_b[sl] = dst_c[bsl]
                return c3

            lax.fori_loop(0, _CH // 16, prep, 0)
            pltpu.async_copy(tab_h.at[idx_b], rows_b, sem).wait()

            def scale_g(g, c3):
                sl = pl.ds(g * 16, 16)
                sv = src_c[pl.ds(cb + g * 16, 16)]
                dv = dsti_b[sl]
                w = _edge_w(asrc_t, adst_t, sv, dv)
                dnm = plsc.load_gather(den_t, [dv])
                cv = w / (dnm + 1e-16)

                def scale_l(l, c4):
                    e = g * 16 + l
                    bc = _lane(cv, l)

                    def scale_v(v, c5):
                        vs = pl.ds(v * 16, 16)
                        rows_b[e, vs] = rows_b[e, vs] * bc
                        return c5

                    lax.fori_loop(0, 8, scale_v, 0)
                    return c4

                lax.fori_loop(0, 16, scale_l, 0)
                return c3

            lax.fori_loop(0, _CH // 16, scale_g, 0)
            pltpu.sync_copy(rows_b, acc_sh.at[dsti_b], add=True)
            return c2

        lax.fori_loop(0, _NCH, chunk, 0)
        return c

    lax.fori_loop(0, _NB, blk, 0)


_MESH = plsc.VectorSubcoreMesh(core_axis_name="c", subcore_axis_name="s")

_SC_PARAMS = pltpu.CompilerParams(
    use_tc_tiling_on_sc=False, needs_layout_passes=False)

_SC_SCRATCH = [
    pltpu.VMEM((_EB,), jnp.int32),
    pltpu.VMEM((_EB,), jnp.int32),
    pltpu.VMEM((_NP,), jnp.float32),
    pltpu.VMEM((_NP,), jnp.float32),
    pltpu.VMEM((_NP,), jnp.float32),
    pltpu.VMEM((16, 128), jnp.float32),
    pltpu.VMEM((128,), jnp.float32),
    pltpu.VMEM((_CH, 128), jnp.float32),
    pltpu.VMEM((_CH,), jnp.int32),
    pltpu.VMEM((_CH,), jnp.int32),
    pltpu.VMEM_SHARED((_NP,), jnp.float32),
    pltpu.VMEM_SHARED((_NP, 128), jnp.float32),
    pltpu.SemaphoreType.DMA,
]


@functools.partial(
    pl.kernel,
    mesh=_MESH,
    out_type=[
        jax.ShapeDtypeStruct((8, _NP, 128), jnp.float32),
        jax.ShapeDtypeStruct((2, 16, _NP), jnp.float32),
    ],
    scratch_types=_SC_SCRATCH,
    compiler_params=_SC_PARAMS,
)
def _l1sc(src_h, dst_h, aTs_h, aTd_h, h1r_h, zacc_h, zvec_h, out_h, dscr_h,
          src_c, dst_c, asrc_t, adst_t, den_t, red_b, redo_c, rows_b,
          idx_b, dsti_b, sh_den, acc_sh, sem):
    cid = lax.axis_index("c")
    sid = lax.axis_index("s")
    eb = sid * _EPB
    r0 = sid * _RPT

    def head_round(hh, c):
        head = cid * 4 + hh
        pltpu.sync_copy(aTs_h.at[head], asrc_t)
        pltpu.sync_copy(aTd_h.at[head], adst_t)
        pltpu.sync_copy(zvec_h, den_t)
        pltpu.sync_copy(zacc_h.at[pl.ds(r0, _RPT)],
                        acc_sh.at[pl.ds(r0, _RPT)])
        plsc.subcore_barrier()
        _stats_pass(eb, src_h, dst_h, src_c, dst_c, asrc_t, adst_t, den_t)
        _reduce_denom(cid, sid, den_t, red_b, redo_c, dscr_h, sh_den)
        _spmm_pass(eb, head, 8, src_h, dst_h, src_c, dst_c, asrc_t, adst_t,
                   den_t, idx_b, dsti_b, rows_b, h1r_h, acc_sh, sem)
        plsc.subcore_barrier()
        pltpu.sync_copy(acc_sh.at[pl.ds(r0, _RPT)],
                        out_h.at[head, pl.ds(r0, _RPT)])
        plsc.subcore_barrier()
        return c

    lax.fori_loop(0, 4, head_round, 0)


@functools.partial(
    pl.kernel,
    mesh=_MESH,
    out_type=[
        jax.ShapeDtypeStruct((2, _NP, 128), jnp.float32),
        jax.ShapeDtypeStruct((2, 16, _NP), jnp.float32),
    ],
    scratch_types=_SC_SCRATCH + [pltpu.VMEM((128,), jnp.float32)],
    compiler_params=_SC_PARAMS,
)
def _l2sc(src_h, dst_h, aTs_h, aTd_h, h2r_h, zacc_h, zvec_h, b2r_h,
          out_h, dscr_h,
          src_c, dst_c, asrc_t, adst_t, den_t, red_b, redo_c, rows_b,
          idx_b, dsti_b, sh_den, acc_sh, sem, b2_t):
    cid = lax.axis_index("c")
    sid = lax.axis_index("s")
    eb = sid * _EPB
    r0 = sid * _RPT
    pltpu.sync_copy(aTs_h, asrc_t)
    pltpu.sync_copy(aTd_h, adst_t)
    pltpu.sync_copy(zvec_h, den_t)
    pltpu.sync_copy(b2r_h.at[cid], b2_t)
    pltpu.sync_copy(zacc_h.at[pl.ds(r0, _RPT)], acc_sh.at[pl.ds(r0, _RPT)])
    plsc.subcore_barrier()
    _stats_pass(eb, src_h, dst_h, src_c, dst_c, asrc_t, adst_t, den_t)
    _reduce_denom(cid, sid, den_t, red_b, redo_c, dscr_h, sh_den)
    _spmm_pass(eb, cid, 2, src_h, dst_h, src_c, dst_c, asrc_t, adst_t,
               den_t, idx_b, dsti_b, rows_b, h2r_h, acc_sh, sem)
    plsc.subcore_barrier()

    def slab(j, c2):
        rr = r0 + j * _CH
        pltpu.sync_copy(acc_sh.at[pl.ds(rr, _CH)], rows_b)

        def rowf(r, c3):
            def colf(v, c4):
                sl = pl.ds(v * 16, 16)
                rows_b[r, sl] = rows_b[r, sl] + b2_t[sl]
                return c4

            lax.fori_loop(0, 8, colf, 0)
            return c3

        lax.fori_loop(0, _CH, rowf, 0)
        pltpu.sync_copy(rows_b, out_h.at[cid, pl.ds(rr, _CH)])
        return c2

    lax.fori_loop(0, _RPT // _CH, slab, 0)


def kernel(x, edge_index, W1, att_src1, att_dst1, b1,
           W2, att_src2, att_dst2, b2):
    src = edge_index[:, 0].astype(jnp.int32)
    dst = edge_index[:, 1].astype(jnp.int32)

    eye8 = jnp.eye(8, dtype=jnp.float32)
    A_src1 = (att_src1[:, :, None] * eye8[:, None, :]).reshape(1024, 8)
    A_dst1 = (att_dst1[:, :, None] * eye8[:, None, :]).reshape(1024, 8)
    A1 = jnp.concatenate([A_src1, A_dst1], axis=1)
    A2 = jnp.zeros((256, 8), jnp.float32)
    A2 = A2.at[:, 0].set(att_src2[0]).at[:, 1].set(att_dst2[0])

    h1, a1 = _proj1(x, W1, A1)
    aT1 = jnp.pad(a1.T, ((0, 0), (0, _NP - _N)))
    aTs1 = aT1[:8]
    aTd1 = aT1[8:]

    zacc = jnp.zeros((_NP, 128), jnp.float32)
    zvec = jnp.zeros((_NP,), jnp.float32)

    out1h, _ = _l1sc(src, dst, aTs1, aTd1, h1.reshape(8 * _N, 128),
                     zacc, zvec)

    h2, a2 = _proj2(out1h[:, :_N], b1.reshape(8, 128),
                    W2.reshape(8, 128, 256), A2)
    aT2 = jnp.pad(a2[:, :2].T, ((0, 0), (0, _NP - _N)))

    out2p, _ = _l2sc(src, dst, aT2[0], aT2[1], h2.reshape(2 * _N, 128),
                     zacc, zvec, b2.reshape(2, 128))
    return jnp.concatenate([out2p[0, :_N], out2p[1, :_N]], axis=1)


# speedup vs baseline: 21.2952x; 1.3448x over previous
import functools

import jax
import jax.numpy as jnp
from jax import lax
from jax.experimental import pallas as pl
from jax.experimental.pallas import tpu as pltpu
from jax.experimental.pallas import tpu_sc as plsc

_N = 10000
_E = 320000
_NP = 10240
_RPT = _NP // 16
_EPB = _E // 16
_EB = 800
_NB = _EPB // _EB
_CH = 32
_CPB = _EB // _CH
_TOT = _NB * _CPB


def _mm_body(x_ref, w_ref, a_ref, h_ref, o_ref):
    h = jnp.dot(x_ref[...], w_ref[...], preferred_element_type=jnp.float32)
    h_ref[...] = h
    o_ref[...] = jnp.dot(h, a_ref[...], preferred_element_type=jnp.float32)


def _proj1(x, W1, A1):
    tn = 400
    return pl.pallas_call(
        _mm_body,
        grid=(_N // tn,),
        in_specs=[
            pl.BlockSpec((tn, 128), lambda i: (i, 0)),
            pl.BlockSpec((128, 1024), lambda i: (0, 0)),
            pl.BlockSpec((1024, 16), lambda i: (0, 0)),
        ],
        out_specs=[
            pl.BlockSpec((tn, 1024), lambda i: (i, 0)),
            pl.BlockSpec((tn, 16), lambda i: (i, 0)),
        ],
        out_shape=[
            jax.ShapeDtypeStruct((_N, 1024), jnp.float32),
            jax.ShapeDtypeStruct((_N, 16), jnp.float32),
        ],
    )(x, W1, A1)


def _mm2_body(o1_ref, b1_ref, w2_ref, a2_ref, h2_ref, ao_ref):
    acc = jnp.zeros((o1_ref.shape[1], 256), jnp.float32)
    for hh in range(8):
        v = o1_ref[hh] + b1_ref[hh]
        xh = jnp.where(v > 0, v, jnp.exp(v) - 1.0)
        acc += jnp.dot(xh, w2_ref[hh], preferred_element_type=jnp.float32)
    h2_ref[...] = acc
    ao_ref[...] = jnp.dot(acc, a2_ref[...], preferred_element_type=jnp.float32)


def _proj2(out1h, b1r, W2r, A2):
    tn = 400
    return pl.pallas_call(
        _mm2_body,
        grid=(_N // tn,),
        in_specs=[
            pl.BlockSpec((8, tn, 128), lambda i: (0, i, 0)),
            pl.BlockSpec((8, 128), lambda i: (0, 0)),
            pl.BlockSpec((8, 128, 256), lambda i: (0, 0, 0)),
            pl.BlockSpec((256, 8), lambda i: (0, 0)),
        ],
        out_specs=[
            pl.BlockSpec((tn, 256), lambda i: (i, 0)),
            pl.BlockSpec((tn, 8), lambda i: (i, 0)),
        ],
        out_shape=[
            jax.ShapeDtypeStruct((_N, 256), jnp.float32),
            jax.ShapeDtypeStruct((_N, 8), jnp.float32),
        ],
    )(out1h, b1r, W2r, A2)


_GDN = lax.GatherDimensionNumbers(
    offset_dims=(), collapsed_slice_dims=(0,), start_index_map=(0,))


def _lane(cv, l):
    idx = jnp.broadcast_to(l, (16, 1)).astype(jnp.int32)
    return lax.gather(cv, idx, _GDN, (1,),
                      mode=lax.GatherScatterMode.PROMISE_IN_BOUNDS)


def _edge_w(asrc_t, adst_t, sv, dv):
    av = plsc.load_gather(asrc_t, [sv]) + plsc.load_gather(adst_t, [dv])
    av = jnp.where(av >= 0, av, av * 0.2)
    return jnp.exp(av)


def _blk_issue(src_h, dst_h, src_c, dst_c, eb, b, buf, sem):
    o = eb + b * _EB
    pltpu.async_copy(src_h.at[pl.ds(o, _EB)], src_c.at[buf], sem)
    pltpu.async_copy(dst_h.at[pl.ds(o, _EB)], dst_c.at[buf], sem)


def _blk_wait(src_h, dst_h, src_c, dst_c, eb, b, buf, sem):
    o = eb + b * _EB
    pltpu.make_async_copy(src_h.at[pl.ds(o, _EB)], src_c.at[buf], sem).wait()
    pltpu.make_async_copy(dst_h.at[pl.ds(o, _EB)], dst_c.at[buf], sem).wait()


def _stats_pass(eb, src_h, dst_h, src_c, dst_c, asrc_t, adst_t, den_t,
                bs0, bs1):
    _blk_issue(src_h, dst_h, src_c, dst_c, eb, 0, 0, bs0)

    def blk(b, c):
        qb = b % 2

        @pl.when(qb == 0)
        def _():
            _blk_wait(src_h, dst_h, src_c, dst_c, eb, b, 0, bs0)

        @pl.when(qb == 1)
        def _():
            _blk_wait(src_h, dst_h, src_c, dst_c, eb, b, 1, bs1)

        @pl.when((b < _NB - 1) & (qb == 1))
        def _():
            _blk_issue(src_h, dst_h, src_c, dst_c, eb, b + 1, 0, bs0)

        @pl.when((b < _NB - 1) & (qb == 0))
        def _():
            _blk_issue(src_h, dst_h, src_c, dst_c, eb, b + 1, 1, bs1)

        def grp(i, c2):
            sl = pl.ds(i * 16, 16)
            dv = dst_c[qb, sl]
            w = _edge_w(asrc_t, adst_t, src_c[qb, sl], dv)
            plsc.addupdate_scatter(den_t, [dv], w)
            return c2

        lax.fori_loop(0, _EB // 16, grp, 0)
        return c

    lax.fori_loop(0, _NB, blk, 0)


def _reduce_inv(cid, sid, den_t, red_b, redo_t, dscr_h):
    pltpu.sync_copy(den_t, dscr_h.at[cid, sid])
    plsc.subcore_barrier()
    nb = sid * _RPT
    for j in range(5):
        pltpu.sync_copy(dscr_h.at[cid, :, pl.ds(nb + j * 128, 128)], red_b)

        def red(i, c2, j=j):
            sl = pl.ds(i * 16, 16)
            acc = red_b[0, sl]
            for r in range(1, 16):
                acc = acc + red_b[r, sl]
            redo_t[pl.ds(j * 128 + i * 16, 16)] = acc
            return c2

        lax.fori_loop(0, 8, red, 0)

    def inv(i, c2):
        sl = pl.ds(i * 16, 16)
        redo_t[sl] = 1.0 / (redo_t[sl] + 1e-16)
        return c2

    lax.fori_loop(0, _RPT // 16, inv, 0)


def _spmm_pipe(eb, head_idx, mult, src_h, dst_h, src_c, dst_c, asrc_t,
               adst_t, idx_b, dsti_b, rows_b, tab_h, acc_sh,
               bs0, bs1, gs0, gs1, ss0, ss1):
    def prep(nci, p1, qn):
        np0 = (nci - (nci // _CPB) * _CPB) * _CH
        for g in range(_CH // 16):
            sl = pl.ds(g * 16, 16)
            bsl = pl.ds(np0 + g * 16, 16)
            idx_b[p1, sl] = src_c[qn, bsl] * mult + head_idx
            dsti_b[p1, sl] = dst_c[qn, bsl]

    def gissue(buf, sem):
        pltpu.async_copy(tab_h.at[idx_b.at[buf]],
                         rows_b.at[pl.ds(buf * _CH, _CH)], sem)

    def gwait(buf, sem):
        pltpu.make_async_copy(tab_h.at[idx_b.at[buf]],
                              rows_b.at[pl.ds(buf * _CH, _CH)], sem).wait()

    def sissue(buf, sem):
        pltpu.async_copy(rows_b.at[pl.ds(buf * _CH, _CH)],
                         acc_sh.at[dsti_b.at[buf]], sem, add=True)

    def swait(buf, sem):
        pltpu.make_async_copy(rows_b.at[pl.ds(buf * _CH, _CH)],
                              acc_sh.at[dsti_b.at[buf]], sem).wait()

    pltpu.sync_copy(src_h.at[pl.ds(eb, _EB)], src_c.at[0])
    pltpu.sync_copy(dst_h.at[pl.ds(eb, _EB)], dst_c.at[0])
    prep(0, 0, 0)
    gissue(0, gs0)

    def it(ci, c):
        p = ci % 2
        blk = ci // _CPB
        pos = ci - blk * _CPB
        qb = blk % 2
        nci = ci + 1
        nblk = nci // _CPB
        npos = nci - nblk * _CPB
        qn = nblk % 2

        @pl.when((ci >= 1) & (p == 1))
        def _():
            swait(0, ss0)

        @pl.when((ci >= 1) & (p == 0))
        def _():
            swait(1, ss1)

        @pl.when((pos == 0) & (blk < _NB - 1) & (qb == 1))
        def _():
            _blk_issue(src_h, dst_h, src_c, dst_c, eb, blk + 1, 0, bs0)

        @pl.when((pos == 0) & (blk < _NB - 1) & (qb == 0))
        def _():
            _blk_issue(src_h, dst_h, src_c, dst_c, eb, blk + 1, 1, bs1)

        @pl.when((nci < _TOT) & (npos == 0) & (qn == 0))
        def _():
            _blk_wait(src_h, dst_h, src_c, dst_c, eb, nblk, 0, bs0)

        @pl.when((nci < _TOT) & (npos == 0) & (qn == 1))
        def _():
            _blk_wait(src_h, dst_h, src_c, dst_c, eb, nblk, 1, bs1)

        @pl.when(nci < _TOT)
        def _():
            prep(nci, 1 - p, qn)

        @pl.when((nci < _TOT) & (p == 1))
        def _():
            gissue(0, gs0)

        @pl.when((nci < _TOT) & (p == 0))
        def _():
            gissue(1, gs1)

        @pl.when(p == 0)
        def _():
            gwait(0, gs0)

        @pl.when(p == 1)
        def _():
            gwait(1, gs1)

        for g in range(_CH // 16):
            sv = src_c[qb, pl.ds(pos * _CH + g * 16, 16)]
            dv = dsti_b[p, pl.ds(g * 16, 16)]
            w = _edge_w(asrc_t, adst_t, sv, dv)

            def lanes(l, c4, g=g, w=w):
                e = p * _CH + g * 16 + l
                bc = _lane(w, l)

                def cols(v, c5):
                    vs = pl.ds(v * 16, 16)
                    rows_b[e, vs] = rows_b[e, vs] * bc
                    return c5

                lax.fori_loop(0, 8, cols, 0)
                return c4

            lax.fori_loop(0, 16, lanes, 0)

        @pl.when(p == 0)
        def _():
            sissue(0, ss0)

        @pl.when(p == 1)
        def _():
            sissue(1, ss1)

        return c

    lax.fori_loop(0, _TOT, it, 0)
    swait(0, ss0)


def _copyout(r0, acc_sh, rows_b, redo_t, out_slab, b2_t=None):
    def co(j, c):
        rr = r0 + j * 64
        pltpu.sync_copy(acc_sh.at[pl.ds(rr, 64)], rows_b)
        for g in range(4):
            iv = redo_t[pl.ds(j * 64 + g * 16, 16)]

            def lanes(l, c2, g=g, iv=iv):
                e = g * 16 + l
                bc = _lane(iv, l)

                def cols(v, c3):
                    vs = pl.ds(v * 16, 16)
                    r = rows_b[e, vs] * bc
                    if b2_t is not None:
                        r = r + b2_t[vs]
                    rows_b[e, vs] = r
                    return c3

                lax.fori_loop(0, 8, cols, 0)
                return c2

            lax.fori_loop(0, 16, lanes, 0)
        pltpu.sync_copy(rows_b, out_slab(rr))
        return c

    lax.fori_loop(0, _RPT // 64, co, 0)


_MESH = plsc.VectorSubcoreMesh(core_axis_name="c", subcore_axis_name="s")

_SC_PARAMS = pltpu.CompilerParams(
    use_tc_tiling_on_sc=False, needs_layout_passes=False)

_SC_SCRATCH = [
    pltpu.VMEM((2, _EB), jnp.int32),
    pltpu.VMEM((2, _EB), jnp.int32),
    pltpu.VMEM((_NP,), jnp.float32),
    pltpu.VMEM((_NP,), jnp.float32),
    pltpu.VMEM((_NP,), jnp.float32),
    pltpu.VMEM((16, 128), jnp.float32),
    pltpu.VMEM((_RPT,), jnp.float32),
    pltpu.VMEM((2 * _CH, 128), jnp.float32),
    pltpu.VMEM((2, _CH), jnp.int32),
    pltpu.VMEM((2, _CH), jnp.int32),
    pltpu.VMEM_SHARED((_NP, 128), jnp.float32),
    pltpu.SemaphoreType.DMA,
    pltpu.SemaphoreType.DMA,
    pltpu.SemaphoreType.DMA,
    pltpu.SemaphoreType.DMA,
    pltpu.SemaphoreType.DMA,
    pltpu.SemaphoreType.DMA,
]


@functools.partial(
    pl.kernel,
    mesh=_MESH,
    out_type=[
        jax.ShapeDtypeStruct((8, _NP, 128), jnp.float32),
        jax.ShapeDtypeStruct((2, 16, _NP), jnp.float32),
    ],
    scratch_types=_SC_SCRATCH,
    compiler_params=_SC_PARAMS,
)
def _l1sc(src_h, dst_h, aTs_h, aTd_h, h1r_h, zacc_h, zvec_h, out_h, dscr_h,
          src_c, dst_c, asrc_t, adst_t, den_t, red_b, redo_t, rows_b,
          idx_b, dsti_b, acc_sh, bs0, bs1, gs0, gs1, ss0, ss1):
    cid = lax.axis_index("c")
    sid = lax.axis_index("s")
    eb = sid * _EPB
    r0 = sid * _RPT

    def head_round(hh, c):
        head = cid * 4 + hh
        pltpu.sync_copy(aTs_h.at[head], asrc_t)
        pltpu.sync_copy(aTd_h.at[head], adst_t)
        pltpu.sync_copy(zvec_h, den_t)
        pltpu.sync_copy(zacc_h.at[pl.ds(r0, _RPT)],
                        acc_sh.at[pl.ds(r0, _RPT)])
        plsc.subcore_barrier()
        _stats_pass(eb, src_h, dst_h, src_c, dst_c, asrc_t, adst_t, den_t,
                    bs0, bs1)
        _reduce_inv(cid, sid, den_t, red_b, redo_t, dscr_h)
        _spmm_pipe(eb, head, 8, src_h, dst_h, src_c, dst_c, asrc_t, adst_t,
                   idx_b, dsti_b, rows_b, h1r_h, acc_sh,
                   bs0, bs1, gs0, gs1, ss0, ss1)
        plsc.subcore_barrier()
        _copyout(r0, acc_sh, rows_b, redo_t,
                 lambda rr: out_h.at[head, pl.ds(rr, 64)])
        plsc.subcore_barrier()
        return c

    lax.fori_loop(0, 4, head_round, 0)


@functools.partial(
    pl.kernel,
    mesh=_MESH,
    out_type=[
        jax.ShapeDtypeStruct((2, _NP, 128), jnp.float32),
        jax.ShapeDtypeStruct((2, 16, _NP), jnp.float32),
    ],
    scratch_types=_SC_SCRATCH + [pltpu.VMEM((128,), jnp.float32)],
    compiler_params=_SC_PARAMS,
)
def _l2sc(src_h, dst_h, aTs_h, aTd_h, h2r_h, zacc_h, zvec_h, b2r_h,
          out_h, dscr_h,
          src_c, dst_c, asrc_t, adst_t, den_t, red_b, redo_t, rows_b,
          idx_b, dsti_b, acc_sh, bs0, bs1, gs0, gs1, ss0, ss1, b2_t):
    cid = lax.axis_index("c")
    sid = lax.axis_index("s")
    eb = sid * _EPB
    r0 = sid * _RPT
    pltpu.sync_copy(aTs_h, asrc_t)
    pltpu.sync_copy(aTd_h, adst_t)
    pltpu.sync_copy(zvec_h, den_t)
    pltpu.sync_copy(b2r_h.at[cid], b2_t)
    pltpu.sync_copy(zacc_h.at[pl.ds(r0, _RPT)], acc_sh.at[pl.ds(r0, _RPT)])
    plsc.subcore_barrier()
    _stats_pass(eb, src_h, dst_h, src_c, dst_c, asrc_t, adst_t, den_t,
                bs0, bs1)
    _reduce_inv(cid, sid, den_t, red_b, redo_t, dscr_h)
    _spmm_pipe(eb, cid, 2, src_h, dst_h, src_c, dst_c, asrc_t, adst_t,
               idx_b, dsti_b, rows_b, h2r_h, acc_sh,
               bs0, bs1, gs0, gs1, ss0, ss1)
    plsc.subcore_barrier()
    _copyout(r0, acc_sh, rows_b, redo_t,
             lambda rr: out_h.at[cid, pl.ds(rr, 64)], b2_t)


def kernel(x, edge_index, W1, att_src1, att_dst1, b1,
           W2, att_src2, att_dst2, b2):
    src = edge_index[:, 0].astype(jnp.int32)
    dst = edge_index[:, 1].astype(jnp.int32)

    eye8 = jnp.eye(8, dtype=jnp.float32)
    A_src1 = (att_src1[:, :, None] * eye8[:, None, :]).reshape(1024, 8)
    A_dst1 = (att_dst1[:, :, None] * eye8[:, None, :]).reshape(1024, 8)
    A1 = jnp.concatenate([A_src1, A_dst1], axis=1)
    A2 = jnp.zeros((256, 8), jnp.float32)
    A2 = A2.at[:, 0].set(att_src2[0]).at[:, 1].set(att_dst2[0])

    h1, a1 = _proj1(x, W1, A1)
    aT1 = jnp.pad(a1.T, ((0, 0), (0, _NP - _N)))
    aTs1 = aT1[:8]
    aTd1 = aT1[8:]

    zacc = jnp.zeros((_NP, 128), jnp.float32)
    zvec = jnp.zeros((_NP,), jnp.float32)

    out1h, _ = _l1sc(src, dst, aTs1, aTd1, h1.reshape(8 * _N, 128),
                     zacc, zvec)

    h2, a2 = _proj2(out1h[:, :_N], b1.reshape(8, 128),
                    W2.reshape(8, 128, 256), A2)
    aT2 = jnp.pad(a2[:, :2].T, ((0, 0), (0, _NP - _N)))

    out2p, _ = _l2sc(src, dst, aT2[0], aT2[1], h2.reshape(2 * _N, 128),
                     zacc, zvec, b2.reshape(2, 128))
    return jnp.concatenate([out2p[0, :_N], out2p[1, :_N]], axis=1)
